# baseline (speedup 1.0000x reference)
import functools

import jax
import jax.numpy as jnp
from jax import lax
from jax.experimental import pallas as pl
from jax.experimental.pallas import tpu as pltpu
from jax.experimental.pallas import tpu_sc as plsc

N = 10000
E = 160000
F_IN = 128
HID = 32
HEADS = 4
EDGE_DIM = 16
AQ = 20
AQ_DIM = 8

NC = 2
NS = 16
NW = NC * NS
NPAD = 10240
ROWS_PER_TILE = NPAD // NS


def _edge_sc_kernel(C, H, ACCW, CH):
  CPH = C // H
  NCHUNK = E // CH
  BASE_CH = NCHUNK // NW
  REM_CH = NCHUNK - BASE_CH * NW
  MAXCH = BASE_CH + (1 if REM_CH else 0)
  HALF = (MAXCH + 1) // 2
  mesh = plsc.VectorSubcoreMesh(core_axis_name="c", subcore_axis_name="s")

  @functools.partial(
      pl.kernel,
      out_type=jax.ShapeDtypeStruct((NC, NPAD, ACCW), jnp.float32),
      mesh=mesh,
      compiler_params=pltpu.CompilerParams(
          needs_layout_passes=False, use_tc_tiling_on_sc=False),
      scratch_types=[
          pltpu.VMEM((HALF, CH), jnp.int32),
          pltpu.VMEM((HALF, CH), jnp.int32),
          pltpu.VMEM((CH, C), jnp.float32),
          pltpu.VMEM((CH, C), jnp.float32),
          pltpu.VMEM((CH, C), jnp.float32),
          pltpu.VMEM((CH, ACCW), jnp.float32),
          pltpu.VMEM((16, C), jnp.float32),
          pltpu.VMEM_SHARED((NPAD, ACCW), jnp.float32),
          pltpu.SemaphoreType.DMA,
          pltpu.SemaphoreType.DMA,
          pltpu.SemaphoreType.DMA,
      ],
  )
  def kern(src_h, dst_h, xl_h, xr_h, ea_h, attb_h, out_h,
           idx_s, idx_d, xl_v, xr_v, ea_v, msg_v, att_v, acc_sh,
           sem0, sem1, sem2):
    c_id = lax.axis_index("c")
    s_id = lax.axis_index("s")
    wid = s_id * NC + c_id

    pltpu.sync_copy(attb_h, att_v)

    zv = jnp.zeros((16,), jnp.float32)

    def zrow(r, carry):
      for j in range(ACCW // 16):
        msg_v[r, pl.ds(j * 16, 16)] = zv
      if ACCW % 16:
        msg_v[r, pl.ds(ACCW - 16, 16)] = zv
      return carry

    lax.fori_loop(0, CH, zrow, 0)

    r0 = s_id * ROWS_PER_TILE

    def zacc(i, carry):
      pltpu.sync_copy(msg_v, acc_sh.at[pl.ds(r0 + i * CH, CH)])
      return carry

    lax.fori_loop(0, ROWS_PER_TILE // CH, zacc, 0)
    plsc.subcore_barrier()

    nch = BASE_CH + jnp.where(wid < REM_CH, 1, 0)
    ch0 = wid * BASE_CH + jnp.minimum(wid, REM_CH)
    iot = lax.iota(jnp.int32, 16)

    cpi0 = pltpu.async_copy(src_h.at[pl.ds(ch0, HALF)], idx_s, sem0)
    cpi1 = pltpu.async_copy(dst_h.at[pl.ds(ch0, HALF)], idx_d, sem1)
    cpi0.wait()
    cpi1.wait()
    off2 = ch0 + nch - HALF

    def chunk_body(i, carry):
      @pl.when(i == HALF)
      def _():
        pltpu.sync_copy(src_h.at[pl.ds(off2, HALF)], idx_s)
        pltpu.sync_copy(dst_h.at[pl.ds(off2, HALF)], idx_d)

      j = jnp.where(i < HALF, i, i - (nch - HALF))
      cp0 = pltpu.async_copy(xl_h.at[idx_s.at[j]], xl_v, sem0)
      cp1 = pltpu.async_copy(xr_h.at[idx_d.at[j]], xr_v, sem1)
      cp2 = pltpu.async_copy(ea_h.at[pl.ds((ch0 + i) * CH, CH)], ea_v, sem2)
      cp0.wait()
      cp1.wait()
      cp2.wait()

      def gbody(g, gc):
        rows = g * 16 + iot
        bph = CPH // 16

        def p1(s, accs):
          perm = (iot + s) & 15
          out = []
          for h in range(H):
            acc = accs[h]
            for cb in range(bph):
              c0 = h * bph + cb
              cvec = perm + (c0 * 16)
              xlv = plsc.load_gather(xl_v, [rows, cvec])
              xrv = plsc.load_gather(xr_v, [rows, cvec])
              eav = plsc.load_gather(ea_v, [rows, cvec])
              attv = att_v[s, pl.ds(c0 * 16, 16)]
              t = xlv + xrv + eav
              t = jnp.maximum(t, t * jnp.float32(0.2))
              acc = acc + t * attv
            out.append(acc)
          return tuple(out)

        accs = lax.fori_loop(
            0, 16, p1, tuple(jnp.zeros((16,), jnp.float32) for _ in range(H)),
            unroll=4)
        ws = [jnp.exp(a) for a in accs]
        for h in range(H):
          plsc.store_scatter(msg_v, [rows, jnp.full((16,), C + h, jnp.int32)],
                             ws[h])

        def p2(s, carry):
          perm = (iot + s) & 15
          for h in range(H):
            for cb in range(bph):
              c0 = h * bph + cb
              cvec = perm + (c0 * 16)
              xlv = plsc.load_gather(xl_v, [rows, cvec])
              plsc.store_scatter(msg_v, [rows, cvec], xlv * ws[h])
          return carry

        lax.fori_loop(0, 16, p2, 0, unroll=4)
        return gc

      lax.fori_loop(0, CH // 16, gbody, 0)
      pltpu.sync_copy(msg_v, acc_sh.at[idx_d.at[j]], add=True)
      return carry

    lax.fori_loop(0, nch, chunk_body, 0)
    plsc.subcore_barrier()
    pltpu.sync_copy(acc_sh.at[pl.ds(r0, ROWS_PER_TILE)],
                    out_h.at[c_id, pl.ds(r0, ROWS_PER_TILE)])

  return kern


_edge_l1 = _edge_sc_kernel(HEADS * HID, HEADS, 136, 64)
_edge_l2 = _edge_sc_kernel(HID, 1, 48, 128)


def _prep_nodes_body(x_ref, aqi_ref, aqt_ref, wl_ref, bl_ref, wr_ref,
                     xl_ref, xr_ref):
  x = x_ref[...]
  aqi = aqi_ref[...]
  oh = (aqi == lax.broadcasted_iota(jnp.int32, (1, AQ), 1)).astype(jnp.float32)
  wl = wl_ref[...]
  wr = wr_ref[...]
  aq_wl = jnp.dot(aqt_ref[...], wl[F_IN:, :], preferred_element_type=jnp.float32)
  aq_wr = jnp.dot(aqt_ref[...], wr[F_IN:, :], preferred_element_type=jnp.float32)
  xl_ref[...] = (jnp.dot(x, wl[:F_IN, :], preferred_element_type=jnp.float32)
                 + jnp.dot(oh, aq_wl, preferred_element_type=jnp.float32)
                 + bl_ref[...])
  xr_ref[...] = (jnp.dot(x, wr[:F_IN, :], preferred_element_type=jnp.float32)
                 + jnp.dot(oh, aq_wr, preferred_element_type=jnp.float32))


def _prep_edges_body(ea_ref, we1_ref, we2_ref, o1_ref, o2_ref):
  ea = ea_ref[...]
  o1_ref[...] = jnp.dot(ea, we1_ref[...], preferred_element_type=jnp.float32)
  o2_ref[...] = jnp.dot(ea, we2_ref[...], preferred_element_type=jnp.float32)


def _elu(v):
  return jnp.where(v > 0, v, jnp.exp(jnp.minimum(v, 0.0)) - 1.0)


def _mid_body(acc_ref, bias1_ref, wl2_ref, bl2_ref, wr2_ref, xl2_ref, xr2_ref):
  acc = acc_ref[0, :N] + acc_ref[1, :N]
  msg = acc[:, :HEADS * HID].reshape(N, HEADS, HID)
  den = acc[:, HEADS * HID:HEADS * HID + HEADS].reshape(N, HEADS, 1)
  out1 = (msg / (den + jnp.float32(1e-16))).reshape(N, HEADS * HID)
  h1 = _elu(out1 + bias1_ref[...])
  xl2_ref[...] = jnp.dot(h1, wl2_ref[...], preferred_element_type=jnp.float32) + bl2_ref[...]
  xr2_ref[...] = jnp.dot(h1, wr2_ref[...], preferred_element_type=jnp.float32)


def _final_body(acc_ref, bias2_ref, wv_ref, bv_ref, wo_ref, bo_ref,
                g_ref, b_ref, fw1_ref, fb1_ref, ow_ref, ob_ref, q_ref):
  acc = acc_ref[0, :N] + acc_ref[1, :N]
  msg = acc[:, :HID]
  den = acc[:, HID:HID + 1]
  h2 = _elu(msg / (den + jnp.float32(1e-16)) + bias2_ref[...])
  v = jnp.dot(h2, wv_ref[...], preferred_element_type=jnp.float32) + bv_ref[...]
  attn = jnp.dot(v, wo_ref[...], preferred_element_type=jnp.float32) + bo_ref[...]
  s = h2 + attn
  mu = jnp.mean(s, axis=-1, keepdims=True)
  var = jnp.mean((s - mu) * (s - mu), axis=-1, keepdims=True)
  hf = (s - mu) / jnp.sqrt(var + jnp.float32(1e-5)) * g_ref[...] + b_ref[...]
  xo = jnp.maximum(jnp.dot(hf, fw1_ref[...], preferred_element_type=jnp.float32)
                   + fb1_ref[...], 0.0)
  q_ref[...] = jnp.dot(xo, ow_ref[...], preferred_element_type=jnp.float32) + ob_ref[...]


def _att_diag_table(att, C):
  a = att.reshape(-1)
  c = jnp.arange(C)
  s = jnp.arange(16)
  idx = (c // 16 * 16)[None, :] + ((c % 16)[None, :] + s[:, None]) % 16
  return a[idx]


def kernel(x, edge_attr, physics_inputs, aq_table, Wl1, bl1, Wr1, We1, att1,
           bias1, Wl2, bl2, Wr2, We2, att2, bias2, Wq, Wk, Wv, bq, bk, bv, Wo,
           bo, ln_g, ln_b, pW1, pb1, pW2, pb2, fW1, fb1, oW, ob, edge_index,
           aquifer_idx):
  f32 = jnp.float32
  src = edge_index[0]
  dst = edge_index[1]
  aqi2 = aquifer_idx.reshape(N, 1)

  xl1, xr1 = pl.pallas_call(
      _prep_nodes_body,
      out_shape=[jax.ShapeDtypeStruct((N, HEADS * HID), f32)] * 2,
  )(x, aqi2, aq_table, Wl1, bl1.reshape(1, -1), Wr1)

  EB = 8000
  ea1, ea2 = pl.pallas_call(
      _prep_edges_body,
      grid=(E // EB,),
      in_specs=[
          pl.BlockSpec((EB, EDGE_DIM), lambda i: (i, 0)),
          pl.BlockSpec((EDGE_DIM, HEADS * HID), lambda i: (0, 0)),
          pl.BlockSpec((EDGE_DIM, HID), lambda i: (0, 0)),
      ],
      out_specs=[
          pl.BlockSpec((EB, HEADS * HID), lambda i: (i, 0)),
          pl.BlockSpec((EB, HID), lambda i: (i, 0)),
      ],
      out_shape=[jax.ShapeDtypeStruct((E, HEADS * HID), f32),
                 jax.ShapeDtypeStruct((E, HID), f32)],
  )(edge_attr, We1, We2)

  acc1 = _edge_l1(src.reshape(-1, 64), dst.reshape(-1, 64), xl1, xr1, ea1,
                  _att_diag_table(att1, HEADS * HID))

  xl2, xr2 = pl.pallas_call(
      _mid_body,
      out_shape=[jax.ShapeDtypeStruct((N, HID), f32)] * 2,
      compiler_params=pltpu.CompilerParams(vmem_limit_bytes=100 * 1024 * 1024),
  )(acc1, bias1.reshape(1, -1), Wl2, bl2.reshape(1, -1), Wr2)

  acc2 = _edge_l2(src.reshape(-1, 128), dst.reshape(-1, 128), xl2, xr2, ea2,
                  _att_diag_table(att2, HID))

  q_out = pl.pallas_call(
      _final_body,
      out_shape=jax.ShapeDtypeStruct((N, 3), f32),
  )(acc2, bias2.reshape(1, -1), Wv, bv.reshape(1, -1), Wo, bo.reshape(1, -1),
    ln_g.reshape(1, -1), ln_b.reshape(1, -1), fW1, fb1.reshape(1, -1), oW,
    ob.reshape(1, -1))
  return q_out


# speedup vs baseline: 1.1256x; 1.1256x over previous
import functools

import jax
import jax.numpy as jnp
from jax import lax
from jax.experimental import pallas as pl
from jax.experimental.pallas import tpu as pltpu
from jax.experimental.pallas import tpu_sc as plsc

N = 10000
E = 160000
F_IN = 128
HID = 32
HEADS = 4
EDGE_DIM = 16
AQ = 20
AQ_DIM = 8

NC = 2
NS = 16
NW = NC * NS
NPAD = 10240
ROWS_PER_TILE = NPAD // NS


def _edge_sc_kernel(C, H, ACCW, CH):
  CPH = C // H
  NCHUNK = E // CH
  BASE_CH = NCHUNK // NW
  REM_CH = NCHUNK - BASE_CH * NW
  MAXCH = BASE_CH + (1 if REM_CH else 0)
  HALF = (MAXCH + 1) // 2
  mesh = plsc.VectorSubcoreMesh(core_axis_name="c", subcore_axis_name="s")

  @functools.partial(
      pl.kernel,
      out_type=jax.ShapeDtypeStruct((NC, NPAD, ACCW), jnp.float32),
      mesh=mesh,
      compiler_params=pltpu.CompilerParams(
          needs_layout_passes=False, use_tc_tiling_on_sc=False),
      scratch_types=[
          pltpu.VMEM((HALF, CH), jnp.int32),
          pltpu.VMEM((HALF, CH), jnp.int32),
          pltpu.VMEM((CH, C), jnp.float32),
          pltpu.VMEM((CH, C), jnp.float32),
          pltpu.VMEM((CH, C), jnp.float32),
          pltpu.VMEM((CH, ACCW), jnp.float32),
          pltpu.VMEM((16, C), jnp.float32),
          pltpu.VMEM_SHARED((NPAD, ACCW), jnp.float32),
          pltpu.SemaphoreType.DMA,
          pltpu.SemaphoreType.DMA,
          pltpu.SemaphoreType.DMA,
      ],
  )
  def kern(src_h, dst_h, xl_h, xr_h, ea_h, attb_h, out_h,
           idx_s, idx_d, xl_v, xr_v, ea_v, msg_v, att_v, acc_sh,
           sem0, sem1, sem2):
    c_id = lax.axis_index("c")
    s_id = lax.axis_index("s")
    wid = s_id * NC + c_id

    pltpu.sync_copy(attb_h, att_v)

    zv = jnp.zeros((16,), jnp.float32)

    def zrow(r, carry):
      for j in range(ACCW // 16):
        msg_v[r, pl.ds(j * 16, 16)] = zv
      if ACCW % 16:
        msg_v[r, pl.ds(ACCW - 16, 16)] = zv
      return carry

    lax.fori_loop(0, CH, zrow, 0)

    r0 = s_id * ROWS_PER_TILE

    def zacc(i, carry):
      pltpu.sync_copy(msg_v, acc_sh.at[pl.ds(r0 + i * CH, CH)])
      return carry

    lax.fori_loop(0, ROWS_PER_TILE // CH, zacc, 0)
    plsc.subcore_barrier()

    nch = BASE_CH + jnp.where(wid < REM_CH, 1, 0)
    ch0 = wid * BASE_CH + jnp.minimum(wid, REM_CH)
    iot = lax.iota(jnp.int32, 16)

    cpi0 = pltpu.async_copy(src_h.at[pl.ds(ch0, HALF)], idx_s, sem0)
    cpi1 = pltpu.async_copy(dst_h.at[pl.ds(ch0, HALF)], idx_d, sem1)
    cpi0.wait()
    cpi1.wait()
    off2 = ch0 + nch - HALF

    def chunk_body(i, carry):
      @pl.when(i == HALF)
      def _():
        pltpu.sync_copy(src_h.at[pl.ds(off2, HALF)], idx_s)
        pltpu.sync_copy(dst_h.at[pl.ds(off2, HALF)], idx_d)

      j = jnp.where(i < HALF, i, i - (nch - HALF))
      cp0 = pltpu.async_copy(xl_h.at[idx_s.at[j]], xl_v, sem0)
      cp1 = pltpu.async_copy(xr_h.at[idx_d.at[j]], xr_v, sem1)
      cp2 = pltpu.async_copy(ea_h.at[pl.ds((ch0 + i) * CH, CH)], ea_v, sem2)
      cp0.wait()
      cp1.wait()
      cp2.wait()

      def gbody(g, gc):
        rows = g * 16 + iot
        bph = CPH // 16

        def p1(s, accs):
          perm = (iot + s) & 15
          out = []
          for h in range(H):
            acc = accs[h]
            for cb in range(bph):
              c0 = h * bph + cb
              cvec = perm + (c0 * 16)
              xlv = plsc.load_gather(xl_v, [rows, cvec])
              xrv = plsc.load_gather(xr_v, [rows, cvec])
              eav = plsc.load_gather(ea_v, [rows, cvec])
              attv = att_v[s, pl.ds(c0 * 16, 16)]
              t = xlv + xrv + eav
              t = jnp.maximum(t, t * jnp.float32(0.2))
              acc = acc + t * attv
            out.append(acc)
          return tuple(out)

        accs = lax.fori_loop(
            0, 16, p1, tuple(jnp.zeros((16,), jnp.float32) for _ in range(H)),
            unroll=2)
        ws = [jnp.exp(a) for a in accs]
        for h in range(H):
          plsc.store_scatter(msg_v, [rows, jnp.full((16,), C + h, jnp.int32)],
                             ws[h])

        def p2(s, carry):
          perm = (iot + s) & 15
          for h in range(H):
            for cb in range(bph):
              c0 = h * bph + cb
              cvec = perm + (c0 * 16)
              xlv = plsc.load_gather(xl_v, [rows, cvec])
              plsc.store_scatter(msg_v, [rows, cvec], xlv * ws[h])
          return carry

        lax.fori_loop(0, 16, p2, 0, unroll=2)
        return gc

      lax.fori_loop(0, CH // 16, gbody, 0)
      pltpu.sync_copy(msg_v, acc_sh.at[idx_d.at[j]], add=True)
      return carry

    lax.fori_loop(0, nch, chunk_body, 0)
    plsc.subcore_barrier()
    pltpu.sync_copy(acc_sh.at[pl.ds(r0, ROWS_PER_TILE)],
                    out_h.at[c_id, pl.ds(r0, ROWS_PER_TILE)])

  return kern


_edge_l1 = _edge_sc_kernel(HEADS * HID, HEADS, 136, 64)
_edge_l2 = _edge_sc_kernel(HID, 1, 48, 128)


def _prep_nodes_body(x_ref, aqi_ref, aqt_ref, wl_ref, bl_ref, wr_ref,
                     xl_ref, xr_ref):
  x = x_ref[...]
  aqi = aqi_ref[...]
  oh = (aqi == lax.broadcasted_iota(jnp.int32, (1, AQ), 1)).astype(jnp.float32)
  wl = wl_ref[...]
  wr = wr_ref[...]
  aq_wl = jnp.dot(aqt_ref[...], wl[F_IN:, :], preferred_element_type=jnp.float32)
  aq_wr = jnp.dot(aqt_ref[...], wr[F_IN:, :], preferred_element_type=jnp.float32)
  xl_ref[...] = (jnp.dot(x, wl[:F_IN, :], preferred_element_type=jnp.float32)
                 + jnp.dot(oh, aq_wl, preferred_element_type=jnp.float32)
                 + bl_ref[...])
  xr_ref[...] = (jnp.dot(x, wr[:F_IN, :], preferred_element_type=jnp.float32)
                 + jnp.dot(oh, aq_wr, preferred_element_type=jnp.float32))


def _prep_edges_body(ea_ref, we1_ref, we2_ref, o1_ref, o2_ref):
  ea = ea_ref[...]
  o1_ref[...] = jnp.dot(ea, we1_ref[...], preferred_element_type=jnp.float32)
  o2_ref[...] = jnp.dot(ea, we2_ref[...], preferred_element_type=jnp.float32)


def _elu(v):
  return jnp.where(v > 0, v, jnp.exp(jnp.minimum(v, 0.0)) - 1.0)


def _mid_body(acc_ref, bias1_ref, wl2_ref, bl2_ref, wr2_ref, xl2_ref, xr2_ref):
  acc = acc_ref[0, :N] + acc_ref[1, :N]
  msg = acc[:, :HEADS * HID].reshape(N, HEADS, HID)
  den = acc[:, HEADS * HID:HEADS * HID + HEADS].reshape(N, HEADS, 1)
  out1 = (msg / (den + jnp.float32(1e-16))).reshape(N, HEADS * HID)
  h1 = _elu(out1 + bias1_ref[...])
  xl2_ref[...] = jnp.dot(h1, wl2_ref[...], preferred_element_type=jnp.float32) + bl2_ref[...]
  xr2_ref[...] = jnp.dot(h1, wr2_ref[...], preferred_element_type=jnp.float32)


def _final_body(acc_ref, bias2_ref, wv_ref, bv_ref, wo_ref, bo_ref,
                g_ref, b_ref, fw1_ref, fb1_ref, ow_ref, ob_ref, q_ref):
  acc = acc_ref[0, :N] + acc_ref[1, :N]
  msg = acc[:, :HID]
  den = acc[:, HID:HID + 1]
  h2 = _elu(msg / (den + jnp.float32(1e-16)) + bias2_ref[...])
  v = jnp.dot(h2, wv_ref[...], preferred_element_type=jnp.float32) + bv_ref[...]
  attn = jnp.dot(v, wo_ref[...], preferred_element_type=jnp.float32) + bo_ref[...]
  s = h2 + attn
  mu = jnp.mean(s, axis=-1, keepdims=True)
  var = jnp.mean((s - mu) * (s - mu), axis=-1, keepdims=True)
  hf = (s - mu) / jnp.sqrt(var + jnp.float32(1e-5)) * g_ref[...] + b_ref[...]
  xo = jnp.maximum(jnp.dot(hf, fw1_ref[...], preferred_element_type=jnp.float32)
                   + fb1_ref[...], 0.0)
  q_ref[...] = jnp.dot(xo, ow_ref[...], preferred_element_type=jnp.float32) + ob_ref[...]


def _att_diag_table(att, C):
  a = att.reshape(-1)
  c = jnp.arange(C)
  s = jnp.arange(16)
  idx = (c // 16 * 16)[None, :] + ((c % 16)[None, :] + s[:, None]) % 16
  return a[idx]


def kernel(x, edge_attr, physics_inputs, aq_table, Wl1, bl1, Wr1, We1, att1,
           bias1, Wl2, bl2, Wr2, We2, att2, bias2, Wq, Wk, Wv, bq, bk, bv, Wo,
           bo, ln_g, ln_b, pW1, pb1, pW2, pb2, fW1, fb1, oW, ob, edge_index,
           aquifer_idx):
  f32 = jnp.float32
  src = edge_index[0]
  dst = edge_index[1]
  aqi2 = aquifer_idx.reshape(N, 1)

  xl1, xr1 = pl.pallas_call(
      _prep_nodes_body,
      out_shape=[jax.ShapeDtypeStruct((N, HEADS * HID), f32)] * 2,
  )(x, aqi2, aq_table, Wl1, bl1.reshape(1, -1), Wr1)

  EB = 8000
  ea1, ea2 = pl.pallas_call(
      _prep_edges_body,
      grid=(E // EB,),
      in_specs=[
          pl.BlockSpec((EB, EDGE_DIM), lambda i: (i, 0)),
          pl.BlockSpec((EDGE_DIM, HEADS * HID), lambda i: (0, 0)),
          pl.BlockSpec((EDGE_DIM, HID), lambda i: (0, 0)),
      ],
      out_specs=[
          pl.BlockSpec((EB, HEADS * HID), lambda i: (i, 0)),
          pl.BlockSpec((EB, HID), lambda i: (i, 0)),
      ],
      out_shape=[jax.ShapeDtypeStruct((E, HEADS * HID), f32),
                 jax.ShapeDtypeStruct((E, HID), f32)],
  )(edge_attr, We1, We2)

  acc1 = _edge_l1(src.reshape(-1, 64), dst.reshape(-1, 64), xl1, xr1, ea1,
                  _att_diag_table(att1, HEADS * HID))

  xl2, xr2 = pl.pallas_call(
      _mid_body,
      out_shape=[jax.ShapeDtypeStruct((N, HID), f32)] * 2,
      compiler_params=pltpu.CompilerParams(vmem_limit_bytes=100 * 1024 * 1024),
  )(acc1, bias1.reshape(1, -1), Wl2, bl2.reshape(1, -1), Wr2)

  acc2 = _edge_l2(src.reshape(-1, 128), dst.reshape(-1, 128), xl2, xr2, ea2,
                  _att_diag_table(att2, HID))

  q_out = pl.pallas_call(
      _final_body,
      out_shape=jax.ShapeDtypeStruct((N, 3), f32),
  )(acc2, bias2.reshape(1, -1), Wv, bv.reshape(1, -1), Wo, bo.reshape(1, -1),
    ln_g.reshape(1, -1), ln_b.reshape(1, -1), fW1, fb1.reshape(1, -1), oW,
    ob.reshape(1, -1))
  return q_out
